# baseline (speedup 1.0000x reference)
import jax
import jax.numpy as jnp
from jax.experimental import pallas as pl

B, L = 4096, 64
MOD_IN = 109
K = 6
MOD_OUT = 8
OUT_FEATURES = 256
AA_DIM = OUT_FEATURES - MOD_OUT
VOCAB = 128

RB = 16
R = RB * L


def _body(idx_ref, mod_ref, wa_ref, wb_ref, out_ref):
    idx3 = idx_ref[...]
    iota = jax.lax.broadcasted_iota(jnp.int32, (RB, L, VOCAB), 2)
    one_hot = (idx3[:, :, None] == iota).astype(jnp.bfloat16).reshape(R, VOCAB)
    mod = mod_ref[...].astype(jnp.bfloat16).reshape(R, MOD_IN)
    acc = jnp.dot(one_hot, wa_ref[...], preferred_element_type=jnp.float32)
    acc += jnp.dot(mod, wb_ref[...], preferred_element_type=jnp.float32)
    out_ref[...] = acc.reshape(RB, L, OUT_FEATURES)


def kernel(aa_indices, mod_x, W_mod, aa_table):
    idx = aa_indices.astype(jnp.int32)

    wa = jnp.concatenate(
        [aa_table, jnp.zeros((VOCAB, MOD_OUT), jnp.float32)], axis=1
    ).astype(jnp.bfloat16)
    wb_top = jnp.concatenate(
        [jnp.zeros((K, AA_DIM), jnp.float32), jnp.eye(K, dtype=jnp.float32),
         jnp.zeros((K, OUT_FEATURES - AA_DIM - K), jnp.float32)], axis=1)
    wb_bot = jnp.concatenate(
        [jnp.zeros((MOD_IN - K, AA_DIM + K), jnp.float32), W_mod], axis=1)
    wb = jnp.concatenate([wb_top, wb_bot], axis=0).astype(jnp.bfloat16)

    return pl.pallas_call(
        _body,
        grid=(B // RB,),
        in_specs=[
            pl.BlockSpec((RB, L), lambda i: (i, 0)),
            pl.BlockSpec((RB, L, MOD_IN), lambda i: (i, 0, 0)),
            pl.BlockSpec((VOCAB, OUT_FEATURES), lambda i: (0, 0)),
            pl.BlockSpec((MOD_IN, OUT_FEATURES), lambda i: (0, 0)),
        ],
        out_specs=pl.BlockSpec((RB, L, OUT_FEATURES), lambda i: (i, 0, 0)),
        out_shape=jax.ShapeDtypeStruct((B, L, OUT_FEATURES), jnp.float32),
    )(idx, mod_x, wa, wb)


# speedup vs baseline: 1.4187x; 1.4187x over previous
import jax
import jax.numpy as jnp
from jax.experimental import pallas as pl

B, L = 4096, 64
MOD_IN = 109
K = 6
MOD_OUT = 8
OUT_FEATURES = 256
AA_DIM = OUT_FEATURES - MOD_OUT
VOCAB = 128

RB = 64
R = RB * L


def _body(idx_ref, mod_ref, wa_ref, wb_ref, out_ref):
    idx3 = idx_ref[...]
    iota = jax.lax.broadcasted_iota(jnp.int32, (RB, L, VOCAB), 2)
    one_hot = (idx3[:, :, None] == iota).astype(jnp.bfloat16).reshape(R, VOCAB)
    mod = mod_ref[...].astype(jnp.bfloat16).reshape(R, MOD_IN)
    acc = jnp.dot(one_hot, wa_ref[...], preferred_element_type=jnp.float32)
    acc += jnp.dot(mod, wb_ref[...], preferred_element_type=jnp.float32)
    out_ref[...] = acc.reshape(RB, L, OUT_FEATURES)


def kernel(aa_indices, mod_x, W_mod, aa_table):
    idx = aa_indices.astype(jnp.int32)

    wa = jnp.concatenate(
        [aa_table, jnp.zeros((VOCAB, MOD_OUT), jnp.float32)], axis=1
    ).astype(jnp.bfloat16)
    wb_top = jnp.concatenate(
        [jnp.zeros((K, AA_DIM), jnp.float32), jnp.eye(K, dtype=jnp.float32),
         jnp.zeros((K, OUT_FEATURES - AA_DIM - K), jnp.float32)], axis=1)
    wb_bot = jnp.concatenate(
        [jnp.zeros((MOD_IN - K, AA_DIM + K), jnp.float32), W_mod], axis=1)
    wb = jnp.concatenate([wb_top, wb_bot], axis=0).astype(jnp.bfloat16)

    return pl.pallas_call(
        _body,
        grid=(B // RB,),
        in_specs=[
            pl.BlockSpec((RB, L), lambda i: (i, 0)),
            pl.BlockSpec((RB, L, MOD_IN), lambda i: (i, 0, 0)),
            pl.BlockSpec((VOCAB, OUT_FEATURES), lambda i: (0, 0)),
            pl.BlockSpec((MOD_IN, OUT_FEATURES), lambda i: (0, 0)),
        ],
        out_specs=pl.BlockSpec((RB, L, OUT_FEATURES), lambda i: (i, 0, 0)),
        out_shape=jax.ShapeDtypeStruct((B, L, OUT_FEATURES), jnp.float32),
    )(idx, mod_x, wa, wb)


# speedup vs baseline: 1.4968x; 1.0550x over previous
import jax
import jax.numpy as jnp
from jax.experimental import pallas as pl

B, L = 4096, 64
MOD_IN = 109
K = 6
MOD_OUT = 8
OUT_FEATURES = 256
AA_DIM = OUT_FEATURES - MOD_OUT
VOCAB = 128

RB = 128
R = RB * L


def _body(idx_ref, mod_ref, wa_ref, wb_ref, out_ref):
    idx3 = idx_ref[...]
    iota = jax.lax.broadcasted_iota(jnp.int32, (RB, L, VOCAB), 2)
    one_hot = (idx3[:, :, None] == iota).astype(jnp.bfloat16).reshape(R, VOCAB)
    mod = mod_ref[...].astype(jnp.bfloat16).reshape(R, MOD_IN)
    acc = jnp.dot(one_hot, wa_ref[...], preferred_element_type=jnp.float32)
    acc += jnp.dot(mod, wb_ref[...], preferred_element_type=jnp.float32)
    out_ref[...] = acc.reshape(RB, L, OUT_FEATURES)


def kernel(aa_indices, mod_x, W_mod, aa_table):
    idx = aa_indices.astype(jnp.int32)

    wa = jnp.concatenate(
        [aa_table, jnp.zeros((VOCAB, MOD_OUT), jnp.float32)], axis=1
    ).astype(jnp.bfloat16)
    wb_top = jnp.concatenate(
        [jnp.zeros((K, AA_DIM), jnp.float32), jnp.eye(K, dtype=jnp.float32),
         jnp.zeros((K, OUT_FEATURES - AA_DIM - K), jnp.float32)], axis=1)
    wb_bot = jnp.concatenate(
        [jnp.zeros((MOD_IN - K, AA_DIM + K), jnp.float32), W_mod], axis=1)
    wb = jnp.concatenate([wb_top, wb_bot], axis=0).astype(jnp.bfloat16)

    return pl.pallas_call(
        _body,
        grid=(B // RB,),
        in_specs=[
            pl.BlockSpec((RB, L), lambda i: (i, 0)),
            pl.BlockSpec((RB, L, MOD_IN), lambda i: (i, 0, 0)),
            pl.BlockSpec((VOCAB, OUT_FEATURES), lambda i: (0, 0)),
            pl.BlockSpec((MOD_IN, OUT_FEATURES), lambda i: (0, 0)),
        ],
        out_specs=pl.BlockSpec((RB, L, OUT_FEATURES), lambda i: (i, 0, 0)),
        out_shape=jax.ShapeDtypeStruct((B, L, OUT_FEATURES), jnp.float32),
    )(idx, mod_x, wa, wb)


# speedup vs baseline: 1.5155x; 1.0125x over previous
import jax
import jax.numpy as jnp
from jax.experimental import pallas as pl

B, L = 4096, 64
MOD_IN = 109
K = 6
MOD_OUT = 8
OUT_FEATURES = 256
AA_DIM = OUT_FEATURES - MOD_OUT
VOCAB = 128

RB = 256
R = RB * L


def _body(idx_ref, mod_ref, wa_ref, wb_ref, out_ref):
    idx3 = idx_ref[...]
    iota = jax.lax.broadcasted_iota(jnp.int32, (RB, L, VOCAB), 2)
    one_hot = (idx3[:, :, None] == iota).astype(jnp.bfloat16).reshape(R, VOCAB)
    mod = mod_ref[...].astype(jnp.bfloat16).reshape(R, MOD_IN)
    acc = jnp.dot(one_hot, wa_ref[...], preferred_element_type=jnp.float32)
    acc += jnp.dot(mod, wb_ref[...], preferred_element_type=jnp.float32)
    out_ref[...] = acc.reshape(RB, L, OUT_FEATURES)


def kernel(aa_indices, mod_x, W_mod, aa_table):
    idx = aa_indices.astype(jnp.int32)

    wa = jnp.concatenate(
        [aa_table, jnp.zeros((VOCAB, MOD_OUT), jnp.float32)], axis=1
    ).astype(jnp.bfloat16)
    wb_top = jnp.concatenate(
        [jnp.zeros((K, AA_DIM), jnp.float32), jnp.eye(K, dtype=jnp.float32),
         jnp.zeros((K, OUT_FEATURES - AA_DIM - K), jnp.float32)], axis=1)
    wb_bot = jnp.concatenate(
        [jnp.zeros((MOD_IN - K, AA_DIM + K), jnp.float32), W_mod], axis=1)
    wb = jnp.concatenate([wb_top, wb_bot], axis=0).astype(jnp.bfloat16)

    return pl.pallas_call(
        _body,
        grid=(B // RB,),
        in_specs=[
            pl.BlockSpec((RB, L), lambda i: (i, 0)),
            pl.BlockSpec((RB, L, MOD_IN), lambda i: (i, 0, 0)),
            pl.BlockSpec((VOCAB, OUT_FEATURES), lambda i: (0, 0)),
            pl.BlockSpec((MOD_IN, OUT_FEATURES), lambda i: (0, 0)),
        ],
        out_specs=pl.BlockSpec((RB, L, OUT_FEATURES), lambda i: (i, 0, 0)),
        out_shape=jax.ShapeDtypeStruct((B, L, OUT_FEATURES), jnp.float32),
    )(idx, mod_x, wa, wb)
